# Initial kernel scaffold; baseline (speedup 1.0000x reference)
import jax
import jax.numpy as jnp
from jax.experimental import pallas as pl


def kernel(f, M01, M02, num_edges, idx_ji, idx_ki, W1, b1, W2, b2, Ws01, bs01, Ws02, bs02, Ws, bs, Wout, bout):
    raise NotImplementedError("write your pallas kernel here")


# speedup vs baseline: 1.5658x; 1.5658x over previous
import functools

import jax
import jax.numpy as jnp
from jax import lax
from jax.experimental import pallas as pl
from jax.experimental.pallas import tpu as pltpu
from jax.experimental.pallas import tpu_sc as plsc

F = 128
E = 160000
T = 480000


_BE = 1000


def _k1_body(f_ref, w1t_ref, b1_ref, w2t_ref, b2_ref, h_ref):
    x = f_ref[...]
    x = jax.nn.silu(jnp.dot(x, w1t_ref[...], preferred_element_type=jnp.float32)
                    + b1_ref[...])
    x = jax.nn.silu(jnp.dot(x, w2t_ref[...], preferred_element_type=jnp.float32)
                    + b2_ref[...])
    a = lax.bitcast_convert_type(x[:, :F].astype(jnp.bfloat16),
                                 jnp.int16).astype(jnp.int32) & 0xFFFF
    b = lax.bitcast_convert_type(x[:, F:].astype(jnp.bfloat16),
                                 jnp.int16).astype(jnp.int32)
    h_ref[...] = a | (b << 16)


def _k1(f, W1t, b1, W2t, b2):
    n = E // _BE
    return pl.pallas_call(
        _k1_body,
        grid=(n,),
        in_specs=[
            pl.BlockSpec((_BE, 2 * F), lambda i: (i, 0)),
            pl.BlockSpec((2 * F, 2 * F), lambda i: (0, 0)),
            pl.BlockSpec((1, 2 * F), lambda i: (0, 0)),
            pl.BlockSpec((2 * F, 2 * F), lambda i: (0, 0)),
            pl.BlockSpec((1, 2 * F), lambda i: (0, 0)),
        ],
        out_specs=pl.BlockSpec((_BE, F), lambda i: (i, 0)),
        out_shape=jax.ShapeDtypeStruct((E, F), jnp.int32),
    )(f, W1t, b1, W2t, b2)


_B2 = 40
_NB2 = T // (32 * _B2)


def _k2_body(h_hbm, ji_hbm, ki_hbm, gki_hbm, gji_hbm,
             ji0, ji1, ji2, ki0, ki1, ki2,
             hki0, hki1, hki2, hji0, hji1, hji2,
             sx0, sx1, sx2, si0, si1, si2, so0, so1, so2):
    wid = lax.axis_index("s") * 2 + lax.axis_index("c")
    base = wid * (T // 32)
    ji = (ji0, ji1, ji2)
    ki = (ki0, ki1, ki2)
    hki = (hki0, hki1, hki2)
    hji = (hji0, hji1, hji2)
    sx = (sx0, sx1, sx2)
    si = (si0, si1, si2)
    so = (so0, so1, so2)

    def issue_idx(k, s):
        rb = base + k * _B2
        pltpu.async_copy(ji_hbm.at[pl.ds(rb, _B2)], ji[s], sx[s])
        pltpu.async_copy(ki_hbm.at[pl.ds(rb, _B2)], ki[s], sx[s])

    def drain_idx(s):
        pltpu.make_async_copy(ji_hbm.at[pl.ds(0, _B2)], ji[s], sx[s]).wait()
        pltpu.make_async_copy(ki_hbm.at[pl.ds(0, _B2)], ki[s], sx[s]).wait()

    def issue_in(k, s):
        pltpu.async_copy(h_hbm.at[ki[s]], hki[s], si[s])
        pltpu.async_copy(h_hbm.at[ji[s]], hji[s], si[s])

    def drain_in(s):
        pltpu.make_async_copy(h_hbm.at[ki[s]], hki[s], si[s]).wait()
        pltpu.make_async_copy(h_hbm.at[ji[s]], hji[s], si[s]).wait()

    def issue_out(k, s):
        rb = base + k * _B2
        pltpu.async_copy(hki[s], gki_hbm.at[pl.ds(rb, _B2)], so[s])
        pltpu.async_copy(hji[s], gji_hbm.at[pl.ds(rb, _B2)], so[s])

    def drain_out(s):
        pltpu.make_async_copy(hki[s], gki_hbm.at[pl.ds(0, _B2)], so[s]).wait()
        pltpu.make_async_copy(hji[s], gji_hbm.at[pl.ds(0, _B2)], so[s]).wait()

    pltpu.sync_copy(ji_hbm.at[pl.ds(base, _B2)], ji[0])
    pltpu.sync_copy(ki_hbm.at[pl.ds(base, _B2)], ki[0])
    issue_in(0, 0)
    issue_idx(1, 1)
    issue_idx(2, 2)
    drain_idx(1)
    issue_in(1, 1)

    def step(k3, carry):
        for b in range(3):
            k = 3 * k3 + b
            s = b
            drain_in(s)
            issue_out(k, s)

            @pl.when(k + 2 < _NB2)
            def _():
                s2 = (b + 2) % 3
                drain_idx(s2)

                @pl.when(k >= 1)
                def _():
                    drain_out(s2)
                issue_in(k + 2, s2)

            @pl.when(k + 3 < _NB2)
            def _():
                issue_idx(k + 3, s)
        return carry

    lax.fori_loop(0, _NB2 // 3, step, 0)
    drain_out(0)
    drain_out(1)
    drain_out(2)


def _k2(h, ji, ki):
    mesh = plsc.VectorSubcoreMesh(core_axis_name="c", subcore_axis_name="s")
    idx_t = pltpu.VMEM((_B2,), jnp.int32)
    g_t = pltpu.VMEM((_B2, F), jnp.int32)
    sem = pltpu.SemaphoreType.DMA
    kern = functools.partial(
        pl.kernel,
        mesh=mesh,
        compiler_params=pltpu.CompilerParams(needs_layout_passes=False),
        out_type=[jax.ShapeDtypeStruct((T, F), jnp.int32),
                  jax.ShapeDtypeStruct((T, F), jnp.int32)],
        scratch_types=(
            [idx_t] * 6 + [g_t] * 6 + [sem] * 9
        ),
    )(_k2_body)
    return kern(h, ji, ki)


_BT = 1000


def _k3_body(gki_ref, gji_ref, m01_ref, m02_ref,
             ws01t, bs01, ws02t, bs02, wsat, wsbt, bs, woutt, c_ref):
    gki = gki_ref[...]
    gji = gji_ref[...]
    maskh = jnp.int32(-65536)
    aki = lax.bitcast_convert_type(gki << 16, jnp.float32)
    aji = lax.bitcast_convert_type(gji << 16, jnp.float32)
    bki = lax.bitcast_convert_type(gki & maskh, jnp.float32)
    bji = lax.bitcast_convert_type(gji & maskh, jnp.float32)
    p01 = aki * m01_ref[...] * aji
    p02 = bki * m02_ref[...] * bji
    s01 = jax.nn.silu(jnp.dot(p01, ws01t[...], preferred_element_type=jnp.float32)
                      + bs01[...])
    s02 = jax.nn.silu(jnp.dot(p02, ws02t[...], preferred_element_type=jnp.float32)
                      + bs02[...])
    s = (jnp.dot(s01, wsat[...], preferred_element_type=jnp.float32)
         + jnp.dot(s02, wsbt[...], preferred_element_type=jnp.float32)
         + bs[...])
    c_ref[...] = jnp.dot(jax.nn.silu(s), woutt[...],
                         preferred_element_type=jnp.float32)


def _k3(gki, gji, M01, M02, Ws01t, bs01, Ws02t, bs02, Wsat, Wsbt, bs, Woutt):
    n = T // _BT
    dspec = pl.BlockSpec((_BT, F), lambda i: (i, 0))
    wspec = pl.BlockSpec((F, F), lambda i: (0, 0))
    bspec = pl.BlockSpec((1, F), lambda i: (0, 0))
    return pl.pallas_call(
        _k3_body,
        grid=(n,),
        in_specs=[
            dspec, dspec, dspec, dspec,
            wspec, bspec, wspec, bspec, wspec, wspec, bspec, wspec,
        ],
        out_specs=pl.BlockSpec((_BT, F), lambda i: (i, 0)),
        out_shape=jax.ShapeDtypeStruct((T, F), jnp.float32),
    )(gki, gji, M01, M02, Ws01t, bs01, Ws02t, bs02, Wsat, Wsbt, bs, Woutt)


_CH = 6400
_NCH = (E + _CH - 1) // _CH
_CHL = E - (_NCH - 1) * _CH
_RPT = _CH // 16
_RPTL = _CHL // 16
_TSH = T // 16
_NV = _TSH // 16
_SENT = _CH + 1


def _k4_body(cp_hbm, ji_hbm, bout_hbm, out_hbm,
             idx_sh, flat_pk, rows0, rows1, pos0, pos1, lidx0, lidx1,
             bias_b, sg0, sg1, sa0, sa1, acc):
    cid = lax.axis_index("c")
    sid = lax.axis_index("s")
    rows = (rows0, rows1)
    posb = (pos0, pos1)
    lidxb = (lidx0, lidx1)
    sg = (sg0, sg1)
    sa = (sa0, sa1)

    pltpu.sync_copy(ji_hbm.at[pl.ds(sid * _TSH, _TSH)], idx_sh)
    pltpu.sync_copy(bout_hbm, bias_b.at[0])
    for c in range(8):
        bv = bias_b[0, pl.ds(c * 16, 16)]
        for r in range(1, 16):
            bias_b[r, pl.ds(c * 16, 16)] = bv

    iota = jnp.arange(16, dtype=jnp.int32)
    sent = jnp.full((16,), _SENT, dtype=jnp.int32)

    def chunk(ci, carry):
        chunk_id = 2 * ci + cid
        lo = chunk_id * _CH
        hi = jnp.minimum(lo + _CH, E)

        def init(k, c2):
            pltpu.sync_copy(bias_b, acc.at[pl.ds(sid * _RPT + k * 16, 16)])
            return c2
        lax.fori_loop(0, _RPT // 16, init, 0)
        if _RPT % 16:
            pltpu.sync_copy(
                bias_b.at[pl.ds(0, _RPT % 16)],
                acc.at[pl.ds(sid * _RPT + _RPT - _RPT % 16, _RPT % 16)])

        @pl.when(sid == 15)
        def _():
            pltpu.sync_copy(bias_b.at[pl.ds(0, 8)], acc.at[pl.ds(_CH, 8)])

        plsc.subcore_barrier()

        def filt(v, n):
            idxv = idx_sh[pl.ds(v * 16, 16)]
            m = (idxv >= lo) & (idxv < hi)
            cum = plsc.cumsum(m.astype(jnp.int32))
            wpos = jnp.where(m, n + cum - 1, _TSH + 64 + iota)
            pk = ((iota + (sid * _TSH + v * 16)) << 13) | (idxv - lo)
            plsc.store_scatter(flat_pk, [wpos], pk)
            return n + cum[15]

        n = lax.fori_loop(0, _NV, filt, jnp.int32(0))
        for t in range(4):
            plsc.store_scatter(flat_pk, [n + t * 16 + iota], sent)

        nb = (n + 63) // 64

        def unpack(j, b):
            for q in range(4):
                pk = flat_pk[pl.ds(j * 64 + q * 16, 16)]
                posb[b][pl.ds(q * 16, 16)] = lax.shift_right_logical(pk, 13)
                lidxb[b][pl.ds(q * 16, 16)] = pk & 8191

        @pl.when(nb > 0)
        def _():
            unpack(0, 0)
            pltpu.async_copy(cp_hbm.at[posb[0]], rows[0], sg[0])

        def batch2(j2, c2):
            for b in range(2):
                j = 2 * j2 + b

                @pl.when(j < nb)
                def _():
                    pltpu.make_async_copy(cp_hbm.at[posb[b]], rows[b],
                                          sg[b]).wait()

                    @pl.when(j >= 1)
                    def _():
                        pltpu.make_async_copy(rows[1 - b], acc.at[lidxb[1 - b]],
                                              sa[1 - b]).wait()

                    @pl.when(j + 1 < nb)
                    def _():
                        unpack(j + 1, 1 - b)
                        pltpu.async_copy(cp_hbm.at[posb[1 - b]], rows[1 - b],
                                         sg[1 - b])
                    pltpu.async_copy(rows[b], acc.at[lidxb[b]], sa[b],
                                     add=True)
            return c2

        lax.fori_loop(0, (nb + 1) // 2, batch2, 0)

        @pl.when((nb & 1) == 1)
        def _():
            pltpu.make_async_copy(rows[0], acc.at[lidxb[0]], sa[0]).wait()

        @pl.when((nb != 0) & ((nb & 1) == 0))
        def _():
            pltpu.make_async_copy(rows[1], acc.at[lidxb[1]], sa[1]).wait()

        plsc.subcore_barrier()

        @pl.when(chunk_id < _NCH - 1)
        def _():
            pltpu.sync_copy(acc.at[pl.ds(sid * _RPT, _RPT)],
                            out_hbm.at[pl.ds(lo + sid * _RPT, _RPT)])

        @pl.when(chunk_id == _NCH - 1)
        def _():
            pltpu.sync_copy(acc.at[pl.ds(sid * _RPTL, _RPTL)],
                            out_hbm.at[pl.ds(lo + sid * _RPTL, _RPTL)])

        plsc.subcore_barrier()
        return carry

    ntrip = jnp.where(cid == 0, (_NCH + 1) // 2, _NCH // 2)
    lax.fori_loop(0, ntrip, chunk, 0)


def _k4(cp, ji, bout):
    mesh = plsc.VectorSubcoreMesh(core_axis_name="c", subcore_axis_name="s")
    kern = functools.partial(
        pl.kernel,
        mesh=mesh,
        compiler_params=pltpu.CompilerParams(needs_layout_passes=False),
        out_type=jax.ShapeDtypeStruct((E, F), jnp.float32),
        scratch_types=[
            pltpu.VMEM((_TSH,), jnp.int32),
            pltpu.VMEM((_TSH + 80, ), jnp.int32),
            pltpu.VMEM((64, F), jnp.float32),
            pltpu.VMEM((64, F), jnp.float32),
            pltpu.VMEM((64,), jnp.int32),
            pltpu.VMEM((64,), jnp.int32),
            pltpu.VMEM((64,), jnp.int32),
            pltpu.VMEM((64,), jnp.int32),
            pltpu.VMEM((16, F), jnp.float32),
            pltpu.SemaphoreType.DMA, pltpu.SemaphoreType.DMA,
            pltpu.SemaphoreType.DMA, pltpu.SemaphoreType.DMA,
            pltpu.VMEM_SHARED((_CH + 8, F), jnp.float32),
        ],
    )(_k4_body)
    return kern(cp, ji, bout)


def kernel(f, M01, M02, num_edges, idx_ji, idx_ki,
           W1, b1, W2, b2, Ws01, bs01, Ws02, bs02, Ws, bs, Wout, bout):
    ji = idx_ji.astype(jnp.int32)
    ki = idx_ki.astype(jnp.int32)
    h = _k1(f, W1.T, b1.reshape(1, -1), W2.T, b2.reshape(1, -1))
    gki, gji = _k2(h, ji, ki)
    cp = _k3(gki, gji, M01, M02,
             Ws01.T, bs01.reshape(1, -1), Ws02.T, bs02.reshape(1, -1),
             Ws[:, :F].T, Ws[:, F:].T, bs.reshape(1, -1), Wout.T)
    return _k4(cp, ji, bout)


# speedup vs baseline: 1.5725x; 1.0043x over previous
import functools

import jax
import jax.numpy as jnp
from jax import lax
from jax.experimental import pallas as pl
from jax.experimental.pallas import tpu as pltpu
from jax.experimental.pallas import tpu_sc as plsc

F = 128
E = 160000
T = 480000


_BE = 1000


def _k1_body(f_ref, w1t_ref, b1_ref, w2t_ref, b2_ref, h_ref):
    x = f_ref[...]
    x = jax.nn.silu(jnp.dot(x, w1t_ref[...], preferred_element_type=jnp.float32)
                    + b1_ref[...])
    x = jax.nn.silu(jnp.dot(x, w2t_ref[...], preferred_element_type=jnp.float32)
                    + b2_ref[...])
    a = lax.bitcast_convert_type(x[:, :F].astype(jnp.bfloat16),
                                 jnp.int16).astype(jnp.int32) & 0xFFFF
    b = lax.bitcast_convert_type(x[:, F:].astype(jnp.bfloat16),
                                 jnp.int16).astype(jnp.int32)
    h_ref[...] = a | (b << 16)


def _k1(f, W1t, b1, W2t, b2):
    n = E // _BE
    return pl.pallas_call(
        _k1_body,
        grid=(n,),
        in_specs=[
            pl.BlockSpec((_BE, 2 * F), lambda i: (i, 0)),
            pl.BlockSpec((2 * F, 2 * F), lambda i: (0, 0)),
            pl.BlockSpec((1, 2 * F), lambda i: (0, 0)),
            pl.BlockSpec((2 * F, 2 * F), lambda i: (0, 0)),
            pl.BlockSpec((1, 2 * F), lambda i: (0, 0)),
        ],
        out_specs=pl.BlockSpec((_BE, F), lambda i: (i, 0)),
        out_shape=jax.ShapeDtypeStruct((E, F), jnp.int32),
    )(f, W1t, b1, W2t, b2)


_B2 = 40
_NB2 = T // (32 * _B2)


def _k2_body(h_hbm, ji_hbm, ki_hbm, gki_hbm, gji_hbm,
             ji0, ji1, ji2, ki0, ki1, ki2,
             hki0, hki1, hki2, hji0, hji1, hji2,
             sx0, sx1, sx2, si0, si1, si2, so0, so1, so2):
    wid = lax.axis_index("s") * 2 + lax.axis_index("c")
    base = wid * (T // 32)
    ji = (ji0, ji1, ji2)
    ki = (ki0, ki1, ki2)
    hki = (hki0, hki1, hki2)
    hji = (hji0, hji1, hji2)
    sx = (sx0, sx1, sx2)
    si = (si0, si1, si2)
    so = (so0, so1, so2)

    def issue_idx(k, s):
        rb = base + k * _B2
        pltpu.async_copy(ji_hbm.at[pl.ds(rb, _B2)], ji[s], sx[s])
        pltpu.async_copy(ki_hbm.at[pl.ds(rb, _B2)], ki[s], sx[s])

    def drain_idx(s):
        pltpu.make_async_copy(ji_hbm.at[pl.ds(0, _B2)], ji[s], sx[s]).wait()
        pltpu.make_async_copy(ki_hbm.at[pl.ds(0, _B2)], ki[s], sx[s]).wait()

    def issue_in(k, s):
        pltpu.async_copy(h_hbm.at[ki[s]], hki[s], si[s])
        pltpu.async_copy(h_hbm.at[ji[s]], hji[s], si[s])

    def drain_in(s):
        pltpu.make_async_copy(h_hbm.at[ki[s]], hki[s], si[s]).wait()
        pltpu.make_async_copy(h_hbm.at[ji[s]], hji[s], si[s]).wait()

    def issue_out(k, s):
        rb = base + k * _B2
        pltpu.async_copy(hki[s], gki_hbm.at[pl.ds(rb, _B2)], so[s])
        pltpu.async_copy(hji[s], gji_hbm.at[pl.ds(rb, _B2)], so[s])

    def drain_out(s):
        pltpu.make_async_copy(hki[s], gki_hbm.at[pl.ds(0, _B2)], so[s]).wait()
        pltpu.make_async_copy(hji[s], gji_hbm.at[pl.ds(0, _B2)], so[s]).wait()

    pltpu.sync_copy(ji_hbm.at[pl.ds(base, _B2)], ji[0])
    pltpu.sync_copy(ki_hbm.at[pl.ds(base, _B2)], ki[0])
    issue_in(0, 0)
    issue_idx(1, 1)
    issue_idx(2, 2)
    drain_idx(1)
    issue_in(1, 1)

    def step(k3, carry):
        for b in range(3):
            k = 3 * k3 + b
            s = b
            drain_in(s)
            issue_out(k, s)

            @pl.when(k + 2 < _NB2)
            def _():
                s2 = (b + 2) % 3
                drain_idx(s2)

                @pl.when(k >= 1)
                def _():
                    drain_out(s2)
                issue_in(k + 2, s2)

            @pl.when(k + 3 < _NB2)
            def _():
                issue_idx(k + 3, s)
        return carry

    lax.fori_loop(0, _NB2 // 3, step, 0)
    drain_out(0)
    drain_out(1)
    drain_out(2)


def _k2(h, ji, ki):
    mesh = plsc.VectorSubcoreMesh(core_axis_name="c", subcore_axis_name="s")
    idx_t = pltpu.VMEM((_B2,), jnp.int32)
    g_t = pltpu.VMEM((_B2, F), jnp.int32)
    sem = pltpu.SemaphoreType.DMA
    kern = functools.partial(
        pl.kernel,
        mesh=mesh,
        compiler_params=pltpu.CompilerParams(needs_layout_passes=False),
        out_type=[jax.ShapeDtypeStruct((T, F), jnp.int32),
                  jax.ShapeDtypeStruct((T, F), jnp.int32)],
        scratch_types=(
            [idx_t] * 6 + [g_t] * 6 + [sem] * 9
        ),
    )(_k2_body)
    return kern(h, ji, ki)


_BT = 1000


def _k3_body(gki_ref, gji_ref, m01_ref, m02_ref,
             ws01t, bs01, ws02t, bs02, wsat, wsbt, bs, woutt, c_ref):
    gki = gki_ref[...]
    gji = gji_ref[...]
    maskh = jnp.int32(-65536)
    aki = lax.bitcast_convert_type(gki << 16, jnp.float32)
    aji = lax.bitcast_convert_type(gji << 16, jnp.float32)
    bki = lax.bitcast_convert_type(gki & maskh, jnp.float32)
    bji = lax.bitcast_convert_type(gji & maskh, jnp.float32)
    p01 = aki * m01_ref[...] * aji
    p02 = bki * m02_ref[...] * bji
    s01 = jax.nn.silu(jnp.dot(p01, ws01t[...], preferred_element_type=jnp.float32)
                      + bs01[...])
    s02 = jax.nn.silu(jnp.dot(p02, ws02t[...], preferred_element_type=jnp.float32)
                      + bs02[...])
    s = (jnp.dot(s01, wsat[...], preferred_element_type=jnp.float32)
         + jnp.dot(s02, wsbt[...], preferred_element_type=jnp.float32)
         + bs[...])
    c_ref[...] = jnp.dot(jax.nn.silu(s), woutt[...],
                         preferred_element_type=jnp.float32)


def _k3(gki, gji, M01, M02, Ws01t, bs01, Ws02t, bs02, Wsat, Wsbt, bs, Woutt):
    n = T // _BT
    dspec = pl.BlockSpec((_BT, F), lambda i: (i, 0))
    wspec = pl.BlockSpec((F, F), lambda i: (0, 0))
    bspec = pl.BlockSpec((1, F), lambda i: (0, 0))
    return pl.pallas_call(
        _k3_body,
        grid=(n,),
        in_specs=[
            dspec, dspec, dspec, dspec,
            wspec, bspec, wspec, bspec, wspec, wspec, bspec, wspec,
        ],
        out_specs=pl.BlockSpec((_BT, F), lambda i: (i, 0)),
        out_shape=jax.ShapeDtypeStruct((T, F), jnp.float32),
    )(gki, gji, M01, M02, Ws01t, bs01, Ws02t, bs02, Wsat, Wsbt, bs, Woutt)


_CH = 6400
_NCH = (E + _CH - 1) // _CH
_CHL = E - (_NCH - 1) * _CH
_RPT = _CH // 16
_RPTL = _CHL // 16
_TSH = T // 16
_NV = _TSH // 16
_SENT = _CH + 1


def _k4_body(cp_hbm, ji_hbm, bout_hbm, out_hbm,
             idx_sh, flat_pk, rows0, rows1, pos0, pos1, lidx0, lidx1,
             bias_b, sg0, sg1, sa0, sa1, acc):
    cid = lax.axis_index("c")
    sid = lax.axis_index("s")
    rows = (rows0, rows1)
    posb = (pos0, pos1)
    lidxb = (lidx0, lidx1)
    sg = (sg0, sg1)
    sa = (sa0, sa1)

    pltpu.sync_copy(ji_hbm.at[pl.ds(sid * _TSH, _TSH)], idx_sh)
    pltpu.sync_copy(bout_hbm, bias_b.at[0])
    for c in range(8):
        bv = bias_b[0, pl.ds(c * 16, 16)]
        for r in range(1, 16):
            bias_b[r, pl.ds(c * 16, 16)] = bv

    iota = jnp.arange(16, dtype=jnp.int32)
    sent = jnp.full((16,), _SENT, dtype=jnp.int32)

    def chunk(ci, carry):
        chunk_id = 2 * ci + cid
        lo = chunk_id * _CH
        hi = jnp.minimum(lo + _CH, E)

        def init(k, c2):
            pltpu.sync_copy(bias_b, acc.at[pl.ds(sid * _RPT + k * 16, 16)])
            return c2
        lax.fori_loop(0, _RPT // 16, init, 0)
        if _RPT % 16:
            pltpu.sync_copy(
                bias_b.at[pl.ds(0, _RPT % 16)],
                acc.at[pl.ds(sid * _RPT + _RPT - _RPT % 16, _RPT % 16)])

        @pl.when(sid == 15)
        def _():
            pltpu.sync_copy(bias_b.at[pl.ds(0, 8)], acc.at[pl.ds(_CH, 8)])

        plsc.subcore_barrier()

        def filt2(v2, n):
            idxv1 = idx_sh[pl.ds(v2 * 32, 16)]
            idxv2 = idx_sh[pl.ds(v2 * 32 + 16, 16)]
            m1 = (idxv1 >= lo) & (idxv1 < hi)
            m2 = (idxv2 >= lo) & (idxv2 < hi)
            cum1 = plsc.cumsum(m1.astype(jnp.int32))
            cum2 = plsc.cumsum(m2.astype(jnp.int32))
            base_t = sid * _TSH + v2 * 32
            wpos1 = jnp.where(m1, n + cum1 - 1, _TSH + 64 + iota)
            pk1 = ((iota + base_t) << 13) | (idxv1 - lo)
            plsc.store_scatter(flat_pk, [wpos1], pk1)
            n1 = n + cum1[15]
            wpos2 = jnp.where(m2, n1 + cum2 - 1, _TSH + 64 + iota)
            pk2 = ((iota + (base_t + 16)) << 13) | (idxv2 - lo)
            plsc.store_scatter(flat_pk, [wpos2], pk2)
            return n1 + cum2[15]

        n = lax.fori_loop(0, _NV // 2, filt2, jnp.int32(0))
        idxv = idx_sh[pl.ds((_NV - 1) * 16, 16)]
        m = (idxv >= lo) & (idxv < hi)
        cum = plsc.cumsum(m.astype(jnp.int32))
        wpos = jnp.where(m, n + cum - 1, _TSH + 64 + iota)
        pk = ((iota + (sid * _TSH + (_NV - 1) * 16)) << 13) | (idxv - lo)
        plsc.store_scatter(flat_pk, [wpos], pk)
        n = n + cum[15]
        for t in range(4):
            plsc.store_scatter(flat_pk, [n + t * 16 + iota], sent)

        nb = (n + 63) // 64

        def unpack(j, b):
            for q in range(4):
                pk = flat_pk[pl.ds(j * 64 + q * 16, 16)]
                posb[b][pl.ds(q * 16, 16)] = lax.shift_right_logical(pk, 13)
                lidxb[b][pl.ds(q * 16, 16)] = pk & 8191

        @pl.when(nb > 0)
        def _():
            unpack(0, 0)
            pltpu.async_copy(cp_hbm.at[posb[0]], rows[0], sg[0])

        def batch2(j2, c2):
            for b in range(2):
                j = 2 * j2 + b

                @pl.when(j < nb)
                def _():
                    pltpu.make_async_copy(cp_hbm.at[posb[b]], rows[b],
                                          sg[b]).wait()

                    @pl.when(j >= 1)
                    def _():
                        pltpu.make_async_copy(rows[1 - b], acc.at[lidxb[1 - b]],
                                              sa[1 - b]).wait()

                    @pl.when(j + 1 < nb)
                    def _():
                        unpack(j + 1, 1 - b)
                        pltpu.async_copy(cp_hbm.at[posb[1 - b]], rows[1 - b],
                                         sg[1 - b])
                    pltpu.async_copy(rows[b], acc.at[lidxb[b]], sa[b],
                                     add=True)
            return c2

        lax.fori_loop(0, (nb + 1) // 2, batch2, 0)

        @pl.when((nb & 1) == 1)
        def _():
            pltpu.make_async_copy(rows[0], acc.at[lidxb[0]], sa[0]).wait()

        @pl.when((nb != 0) & ((nb & 1) == 0))
        def _():
            pltpu.make_async_copy(rows[1], acc.at[lidxb[1]], sa[1]).wait()

        plsc.subcore_barrier()

        @pl.when(chunk_id < _NCH - 1)
        def _():
            pltpu.sync_copy(acc.at[pl.ds(sid * _RPT, _RPT)],
                            out_hbm.at[pl.ds(lo + sid * _RPT, _RPT)])

        @pl.when(chunk_id == _NCH - 1)
        def _():
            pltpu.sync_copy(acc.at[pl.ds(sid * _RPTL, _RPTL)],
                            out_hbm.at[pl.ds(lo + sid * _RPTL, _RPTL)])

        plsc.subcore_barrier()
        return carry

    ntrip = jnp.where(cid == 0, (_NCH + 1) // 2, _NCH // 2)
    lax.fori_loop(0, ntrip, chunk, 0)


def _k4(cp, ji, bout):
    mesh = plsc.VectorSubcoreMesh(core_axis_name="c", subcore_axis_name="s")
    kern = functools.partial(
        pl.kernel,
        mesh=mesh,
        compiler_params=pltpu.CompilerParams(needs_layout_passes=False),
        out_type=jax.ShapeDtypeStruct((E, F), jnp.float32),
        scratch_types=[
            pltpu.VMEM((_TSH,), jnp.int32),
            pltpu.VMEM((_TSH + 80, ), jnp.int32),
            pltpu.VMEM((64, F), jnp.float32),
            pltpu.VMEM((64, F), jnp.float32),
            pltpu.VMEM((64,), jnp.int32),
            pltpu.VMEM((64,), jnp.int32),
            pltpu.VMEM((64,), jnp.int32),
            pltpu.VMEM((64,), jnp.int32),
            pltpu.VMEM((16, F), jnp.float32),
            pltpu.SemaphoreType.DMA, pltpu.SemaphoreType.DMA,
            pltpu.SemaphoreType.DMA, pltpu.SemaphoreType.DMA,
            pltpu.VMEM_SHARED((_CH + 8, F), jnp.float32),
        ],
    )(_k4_body)
    return kern(cp, ji, bout)


def kernel(f, M01, M02, num_edges, idx_ji, idx_ki,
           W1, b1, W2, b2, Ws01, bs01, Ws02, bs02, Ws, bs, Wout, bout):
    ji = idx_ji.astype(jnp.int32)
    ki = idx_ki.astype(jnp.int32)
    h = _k1(f, W1.T, b1.reshape(1, -1), W2.T, b2.reshape(1, -1))
    gki, gji = _k2(h, ji, ki)
    cp = _k3(gki, gji, M01, M02,
             Ws01.T, bs01.reshape(1, -1), Ws02.T, bs02.reshape(1, -1),
             Ws[:, :F].T, Ws[:, F:].T, bs.reshape(1, -1), Wout.T)
    return _k4(cp, ji, bout)
